# baseline (speedup 1.0000x reference)
import functools

import jax
import jax.numpy as jnp
from jax import lax
from jax.experimental import pallas as pl
from jax.experimental.pallas import tpu as pltpu
from jax.experimental.pallas import tpu_sc as plsc

N = 16384
D = 32
DP = 33
C = 1000
HC = 500
CP = 512
DUMMY = CP - 1
NC = 2
NS = 16
R = N // NS
CHUNK = 128
NCHUNK = R // CHUNK
L = 16
G = R // L


def _lanes_f32(val):
    return jnp.full((L,), val, dtype=jnp.float32)


def _sc_body(emb_hbm, tgt_hbm, out_hbm,
             emb_v, tgt_v, tgtr_v, rix_v, ctl_v, sums_l, cnts_l,
             ones_v, zrow_v, zcnt_v, part_v,
             sums_sh, cnts_sh, sem_e, sem_t, sem_s):
    cid = lax.axis_index("c")
    wid = lax.axis_index("s")
    base = wid * R
    zero16 = _lanes_f32(0.0)
    lane = lax.broadcasted_iota(jnp.int32, (L,), 0)

    cp_emb = pltpu.async_copy(emb_hbm.at[pl.ds(base, R), :], emb_v, sem_e)
    cp_tgt = pltpu.async_copy(tgt_hbm.at[wid], tgt_v, sem_t)

    for i in range(CP // NS):
        zrow_v[i, pl.ds(0, L)] = zero16
        zrow_v[i, pl.ds(L, L)] = zero16
    for i in range((CP // NS) // L):
        zcnt_v[pl.ds(i * L, L)] = zero16
    for i in range(CHUNK // L):
        ones_v[pl.ds(i * L, L)] = _lanes_f32(1.0)
    pltpu.sync_copy(zrow_v, sums_sh.at[pl.ds(wid * (CP // NS), CP // NS), :])
    pltpu.sync_copy(zcnt_v, cnts_sh.at[pl.ds(wid * (CP // NS), CP // NS)])
    cp_tgt.wait()

    cbase = cid * HC
    dummy16 = jnp.full((L,), DUMMY, dtype=jnp.int32)
    for i in range(G // L):
        rix_v[pl.ds(i * L, L)] = lane * 0
        ctl_v[pl.ds(i * L, L)] = dummy16
    for i in range(G // L, R // L):
        rix_v[pl.ds(i * L, L)] = lane * 0
        ctl_v[pl.ds(i * L, L)] = dummy16

    offv = lane * 0
    for g in range(G):
        j, col = g >> 3, (g & 7) * L
        tv = tgt_v[j, pl.ds(col, L)]
        tl = tv - cbase
        valid = (tl >= 0) & (tl < HC)
        vi = valid.astype(jnp.int32)
        tlr = jnp.where(valid, tl, DUMMY)
        tgtr_v[j, pl.ds(col, L)] = tlr
        pos = offv + plsc.cumsum(vi) - vi
        rv = g * L + lane
        plsc.store_scatter(rix_v, [pos], rv, mask=valid)
        plsc.store_scatter(ctl_v, [pos], tlr, mask=valid)
        offv = offv + plsc.all_reduce_population_count(valid)
    nvalid = jnp.max(offv)

    cp_emb.wait()
    plsc.subcore_barrier()

    adds = []
    for j in range(NCHUNK):
        idx = tgtr_v.at[j]
        adds.append(pltpu.async_copy(emb_v.at[pl.ds(j * CHUNK, CHUNK), :],
                                     sums_sh.at[idx], sem_s, add=True))
        adds.append(pltpu.async_copy(ones_v, cnts_sh.at[idx], sem_s,
                                     add=True))
    for a in adds:
        a.wait()
    plsc.subcore_barrier()

    pltpu.sync_copy(sums_sh, sums_l)
    pltpu.sync_copy(cnts_sh, cnts_l)

    DSUB = 8

    def group_body(k, acc):
        kv = k * L + lane
        rvk = rix_v[pl.ds(k * L, L)]
        tlv = ctl_v[pl.ds(k * L, L)]
        mrow = kv < offv
        cv = plsc.load_gather(cnts_l, [tlv])
        invc = 1.0 / jnp.maximum(cv, 1.0)

        def dim_body(j, sqs):
            sq0, sq1 = sqs
            d0 = j * DSUB
            for kk in range(DSUB):
                dvv = jnp.full((L,), 0, dtype=jnp.int32) + (d0 + kk)
                ev = plsc.load_gather(emb_v, [rvk, dvv])
                sv = plsc.load_gather(sums_l, [tlv, dvv])
                diff = ev - invc * sv
                if kk & 1:
                    sq1 = sq1 + diff * diff
                else:
                    sq0 = sq0 + diff * diff
            return (sq0, sq1)

        sq0, sq1 = lax.fori_loop(0, D // DSUB, dim_body,
                                 (_lanes_f32(0.0), _lanes_f32(0.0)))
        sq = sq0 + sq1
        i = plsc.bitcast(sq, jnp.int32)
        i = 0x5F3759DF - lax.shift_right_logical(i, 1)
        y = plsc.bitcast(i, jnp.float32)
        for _ in range(3):
            y = y * (1.5 - 0.5 * sq * y * y)
        norm = jnp.where((sq > 0.0) & mrow, sq * y, 0.0)
        return acc + norm * invc

    ngroups = lax.shift_right_logical(nvalid + (L - 1), 4)
    part = lax.fori_loop(0, ngroups, group_body, _lanes_f32(0.0))
    part_v[...] = part
    pltpu.sync_copy(part_v, out_hbm.at[pl.ds((cid * NS + wid) * L, L)])


@jax.jit
def _center_loss_sc(embeddeds, tgt3d):
    mesh = plsc.VectorSubcoreMesh(core_axis_name="c", subcore_axis_name="s",
                                  num_cores=NC, num_subcores=NS)
    f = pl.kernel(
        _sc_body,
        out_type=jax.ShapeDtypeStruct((NC * NS * L,), jnp.float32),
        mesh=mesh,
        compiler_params=pltpu.CompilerParams(use_tc_tiling_on_sc=False,
                                             needs_layout_passes=False),
        scratch_types=[
            pltpu.VMEM((R, D), jnp.float32),
            pltpu.VMEM((NCHUNK, CHUNK), jnp.int32),
            pltpu.VMEM((NCHUNK, CHUNK), jnp.int32),
            pltpu.VMEM((R,), jnp.int32),
            pltpu.VMEM((R,), jnp.int32),
            pltpu.VMEM((CP, D), jnp.float32),
            pltpu.VMEM((CP,), jnp.float32),
            pltpu.VMEM((CHUNK,), jnp.float32),
            pltpu.VMEM((CP // NS, D), jnp.float32),
            pltpu.VMEM((CP // NS,), jnp.float32),
            pltpu.VMEM((L,), jnp.float32),
            pltpu.VMEM_SHARED((CP, D), jnp.float32),
            pltpu.VMEM_SHARED((CP,), jnp.float32),
            pltpu.SemaphoreType.DMA,
            pltpu.SemaphoreType.DMA,
            pltpu.SemaphoreType.DMA,
        ],
    )
    return f(embeddeds, tgt3d)


def kernel(embeddeds, target):
    tgt3d = target.astype(jnp.int32).reshape(NS, NCHUNK, CHUNK)
    out = _center_loss_sc(embeddeds, tgt3d)
    return jnp.sum(out)


# speedup vs baseline: 1.0670x; 1.0670x over previous
import functools

import jax
import jax.numpy as jnp
from jax import lax
from jax.experimental import pallas as pl
from jax.experimental.pallas import tpu as pltpu
from jax.experimental.pallas import tpu_sc as plsc

N = 16384
D = 32
W = D // 2
C = 1000
CP = 1024
NS = 16
R = N // NS
CHUNK = 128
NCHUNK = R // CHUNK
L = 16
OSL = CP // NS


def _lanes_f32(val):
    return jnp.full((L,), val, dtype=jnp.float32)


def _sc_body(emb_hbm, embp_hbm, tgt_hbm, out_hbm,
             emb_v, embp_v, tgt_v, myrows_v, mypk_v, sumsp_l, cnts_l,
             ones_v, zrow_v, zcnt_v, part_v,
             sums_sh, cnts_sh, sumsp_sh, sem_e, sem_p, sem_t, sem_s):
    wid = lax.axis_index("s")
    base = wid * R
    zero16 = _lanes_f32(0.0)
    lane = lax.broadcasted_iota(jnp.int32, (L,), 0)

    cp_emb = pltpu.async_copy(emb_hbm.at[pl.ds(base, R), :], emb_v, sem_e)
    cp_pk = pltpu.async_copy(embp_hbm.at[pl.ds(base, R), :], embp_v, sem_p)
    cp_tgt = pltpu.async_copy(tgt_hbm.at[wid], tgt_v, sem_t)

    for i in range(OSL):
        zrow_v[i, pl.ds(0, L)] = zero16
        zrow_v[i, pl.ds(L, L)] = zero16
    for i in range(OSL // L):
        zcnt_v[pl.ds(i * L, L)] = zero16
    for i in range(CHUNK // L):
        ones_v[pl.ds(i * L, L)] = _lanes_f32(1.0)
    pltpu.sync_copy(zrow_v, sums_sh.at[pl.ds(wid * OSL, OSL), :])
    pltpu.sync_copy(zcnt_v, cnts_sh.at[pl.ds(wid * OSL, OSL)])
    cp_emb.wait()
    cp_tgt.wait()
    plsc.subcore_barrier()

    adds = []
    for j in range(NCHUNK):
        idx = tgt_v.at[j]
        adds.append(pltpu.async_copy(emb_v.at[pl.ds(j * CHUNK, CHUNK), :],
                                     sums_sh.at[idx], sem_s, add=True))
        adds.append(pltpu.async_copy(ones_v, cnts_sh.at[idx], sem_s,
                                     add=True))
    for a in adds:
        a.wait()
    plsc.subcore_barrier()

    pltpu.sync_copy(sums_sh.at[pl.ds(wid * OSL, OSL), :], myrows_v)
    for grp in range(OSL // L):
        rloc = grp * L + lane
        for w in range(W):
            wa = jnp.full((L,), 2 * w, dtype=jnp.int32)
            wb = jnp.full((L,), 2 * w + 1, dtype=jnp.int32)
            ga = plsc.load_gather(myrows_v, [rloc, wa])
            gb = plsc.load_gather(myrows_v, [rloc, wb])
            pk = plsc.pack(ga, gb, format=plsc.PackFormat.INTERLEAVED)
            wi = plsc.bitcast(pk, jnp.int32)
            wv = jnp.full((L,), w, dtype=jnp.int32)
            plsc.store_scatter(mypk_v, [rloc, wv], wi)
    pltpu.sync_copy(mypk_v, sumsp_sh.at[pl.ds(wid * OSL, OSL), :])
    plsc.subcore_barrier()

    pltpu.sync_copy(sumsp_sh, sumsp_l)
    pltpu.sync_copy(cnts_sh, cnts_l)
    cp_pk.wait()

    WSUB = 4

    def group_body(g, acc):
        rv = g * L + lane
        tv = plsc.load_gather(tgt_v, [lax.shift_right_logical(rv, 7),
                                      rv & (CHUNK - 1)])
        cv = plsc.load_gather(cnts_l, [tv])
        invc = 1.0 / jnp.maximum(cv, 1.0)

        def dim_body(j, sqs):
            sq0, sq1 = sqs
            w0 = j * WSUB
            for kk in range(WSUB):
                wv = jnp.full((L,), 0, dtype=jnp.int32) + (w0 + kk)
                ew = plsc.load_gather(embp_v, [rv, wv])
                sw = plsc.load_gather(sumsp_l, [tv, wv])
                ea, eb = plsc.unpack(plsc.bitcast(ew, jnp.bfloat16),
                                     format=plsc.PackFormat.INTERLEAVED)
                sa, sb = plsc.unpack(plsc.bitcast(sw, jnp.bfloat16),
                                     format=plsc.PackFormat.INTERLEAVED)
                d0 = ea - invc * sa
                d1 = eb - invc * sb
                sq0 = sq0 + d0 * d0
                sq1 = sq1 + d1 * d1
            return (sq0, sq1)

        sq0, sq1 = lax.fori_loop(0, W // WSUB, dim_body,
                                 (_lanes_f32(0.0), _lanes_f32(0.0)))
        sq = sq0 + sq1
        i = plsc.bitcast(sq, jnp.int32)
        i = 0x5F3759DF - lax.shift_right_logical(i, 1)
        y = plsc.bitcast(i, jnp.float32)
        for _ in range(3):
            y = y * (1.5 - 0.5 * sq * y * y)
        norm = jnp.where(sq > 0.0, sq * y, 0.0)
        return acc + norm * invc

    part = lax.fori_loop(0, R // L, group_body, _lanes_f32(0.0))
    part_v[...] = part
    pltpu.sync_copy(part_v, out_hbm.at[pl.ds(wid * L, L)])


@jax.jit
def _center_loss_sc(embeddeds, embp, tgt3d):
    mesh = plsc.VectorSubcoreMesh(core_axis_name="c", subcore_axis_name="s",
                                  num_cores=1, num_subcores=NS)
    f = pl.kernel(
        _sc_body,
        out_type=jax.ShapeDtypeStruct((NS * L,), jnp.float32),
        mesh=mesh,
        compiler_params=pltpu.CompilerParams(use_tc_tiling_on_sc=False,
                                             needs_layout_passes=False),
        scratch_types=[
            pltpu.VMEM((R, D), jnp.float32),
            pltpu.VMEM((R, W), jnp.int32),
            pltpu.VMEM((NCHUNK, CHUNK), jnp.int32),
            pltpu.VMEM((OSL, D), jnp.float32),
            pltpu.VMEM((OSL, W), jnp.int32),
            pltpu.VMEM((CP, W), jnp.int32),
            pltpu.VMEM((CP,), jnp.float32),
            pltpu.VMEM((CHUNK,), jnp.float32),
            pltpu.VMEM((OSL, D), jnp.float32),
            pltpu.VMEM((OSL,), jnp.float32),
            pltpu.VMEM((L,), jnp.float32),
            pltpu.VMEM_SHARED((CP, D), jnp.float32),
            pltpu.VMEM_SHARED((CP,), jnp.float32),
            pltpu.VMEM_SHARED((CP, W), jnp.int32),
            pltpu.SemaphoreType.DMA,
            pltpu.SemaphoreType.DMA,
            pltpu.SemaphoreType.DMA,
            pltpu.SemaphoreType.DMA,
        ],
    )
    return f(embeddeds, embp, tgt3d)


def kernel(embeddeds, target):
    tgt3d = target.astype(jnp.int32).reshape(NS, NCHUNK, CHUNK)
    ebf = embeddeds.astype(jnp.bfloat16).reshape(N, W, 2)
    embp = lax.bitcast_convert_type(ebf, jnp.int32)
    out = _center_loss_sc(embeddeds, embp, tgt3d)
    return jnp.sum(out)
